# baseline (speedup 1.0000x reference)
import functools

import jax
import jax.numpy as jnp
from jax import lax
from jax.experimental import pallas as pl
from jax.experimental.pallas import tpu as pltpu
from jax.experimental.pallas import tpu_sc as plsc

_LANES = 16


@functools.lru_cache(maxsize=None)
def _make_sc_add(n_rows: int, seq_len: int, d: int, nc: int, ns: int,
                 chunk: int):
    nw = nc * ns
    assert n_rows % nw == 0
    rows_per_w = n_rows // nw
    assert seq_len % rows_per_w == 0, "worker block must not cross batches"
    assert rows_per_w % chunk == 0
    n_chunks = rows_per_w // chunk
    chunk_elems = chunk * d
    assert chunk_elems % _LANES == 0
    n_vec = chunk_elems // _LANES

    mesh = plsc.VectorSubcoreMesh(
        core_axis_name="c", subcore_axis_name="s",
        num_cores=nc, num_subcores=ns)

    @functools.partial(
        pl.kernel,
        out_type=jax.ShapeDtypeStruct((n_rows * d,), jnp.float32),
        mesh=mesh,
        scratch_types=[
            pltpu.VMEM((chunk_elems,), jnp.float32),
            pltpu.VMEM((chunk_elems,), jnp.float32),
        ],
    )
    def sc_add(x_hbm, pe_hbm, out_hbm, xv, pv):
        wid = lax.axis_index("s") * nc + lax.axis_index("c")
        row_base = wid * rows_per_w
        seq_base = lax.rem(row_base, seq_len)

        @pl.loop(0, n_chunks)
        def _chunk(k):
            x_off = (row_base + k * chunk) * d
            pe_off = (seq_base + k * chunk) * d
            pltpu.sync_copy(x_hbm.at[pl.ds(x_off, chunk_elems)], xv)
            pltpu.sync_copy(pe_hbm.at[pl.ds(pe_off, chunk_elems)], pv)

            @pl.loop(0, n_vec, unroll=8)
            def _add(i):
                o = i * _LANES
                xv[pl.ds(o, _LANES)] = xv[pl.ds(o, _LANES)] + pv[pl.ds(o, _LANES)]

            pltpu.sync_copy(xv, out_hbm.at[pl.ds(x_off, chunk_elems)])

    return sc_add


def kernel(x, pos_embedding):
    b, s, d = x.shape
    info = plsc.get_sparse_core_info()
    fn = _make_sc_add(b * s, s, d, info.num_cores, info.num_subcores, 32)
    out = fn(x.reshape(b * s * d), pos_embedding.reshape(-1))
    return out.reshape(b, s, d)


# speedup vs baseline: 1.6990x; 1.6990x over previous
import functools

import jax
import jax.numpy as jnp
from jax import lax
from jax.experimental import pallas as pl
from jax.experimental.pallas import tpu as pltpu
from jax.experimental.pallas import tpu_sc as plsc

_LANES = 16
_CHUNK_ROWS = 8
_N_IN = 3
_N_OUT = 2


@functools.lru_cache(maxsize=None)
def _make_sc_add(batch: int, seq_len: int, d: int, nc: int, ns: int):
    nw = nc * ns
    assert seq_len % nw == 0
    s_per_w = seq_len // nw
    pe_elems = s_per_w * d
    chunk = min(_CHUNK_ROWS, s_per_w)
    assert s_per_w % chunk == 0
    cpb = s_per_w // chunk
    n_chunks = batch * cpb
    chunk_elems = chunk * d
    assert chunk_elems % _LANES == 0
    n_vec = chunk_elems // _LANES
    n_in = min(_N_IN, n_chunks)
    n_out = min(_N_OUT, n_chunks)

    mesh = plsc.VectorSubcoreMesh(
        core_axis_name="c", subcore_axis_name="s",
        num_cores=nc, num_subcores=ns)

    @functools.partial(
        pl.kernel,
        out_type=jax.ShapeDtypeStruct((batch * seq_len * d,), jnp.float32),
        mesh=mesh,
        scratch_types=(
            [pltpu.VMEM((pe_elems,), jnp.float32)]
            + [pltpu.VMEM((chunk_elems,), jnp.float32) for _ in range(n_in)]
            + [pltpu.VMEM((chunk_elems,), jnp.float32) for _ in range(n_out)]
            + [pltpu.SemaphoreType.DMA for _ in range(n_in + n_out)]
        ),
    )
    def sc_add(x_hbm, pe_hbm, out_hbm, *scratch):
        pv = scratch[0]
        xv = scratch[1:1 + n_in]
        ov = scratch[1 + n_in:1 + n_in + n_out]
        in_sem = scratch[1 + n_in + n_out:1 + 2 * n_in + n_out]
        out_sem = scratch[1 + 2 * n_in + n_out:]

        wid = lax.axis_index("s") * nc + lax.axis_index("c")
        seq_base = wid * s_per_w

        pltpu.sync_copy(pe_hbm.at[pl.ds(seq_base * d, pe_elems)], pv)

        def x_off(t):
            b, c = divmod(t, cpb)
            return (b * seq_len + seq_base + c * chunk) * d

        in_d = [None] * n_chunks
        out_d = [None] * n_chunks
        for t in range(n_in):
            in_d[t] = pltpu.async_copy(
                x_hbm.at[pl.ds(x_off(t), chunk_elems)], xv[t % n_in],
                in_sem[t % n_in])

        for t in range(n_chunks):
            ib = t % n_in
            ob = t % n_out
            in_d[t].wait()
            if t - n_out >= 0:
                out_d[t - n_out].wait()

            pe_base = (t % cpb) * chunk_elems

            @plsc.parallel_loop(0, n_vec, unroll=8)
            def _add(i, _ib=ib, _ob=ob, _pb=pe_base):
                o = i * _LANES
                ov[_ob][pl.ds(o, _LANES)] = (
                    xv[_ib][pl.ds(o, _LANES)] + pv[pl.ds(_pb + o, _LANES)])

            out_d[t] = pltpu.async_copy(
                ov[ob], out_hbm.at[pl.ds(x_off(t), chunk_elems)], out_sem[ob])
            if t + n_in < n_chunks:
                in_d[t + n_in] = pltpu.async_copy(
                    x_hbm.at[pl.ds(x_off(t + n_in), chunk_elems)], xv[ib],
                    in_sem[ib])

        for t in range(max(0, n_chunks - n_out), n_chunks):
            out_d[t].wait()

    return sc_add


def kernel(x, pos_embedding):
    b, s, d = x.shape
    info = plsc.get_sparse_core_info()
    fn = _make_sc_add(b, s, d, info.num_cores, info.num_subcores)
    out = fn(x.reshape(b * s * d), pos_embedding.reshape(-1))
    return out.reshape(b, s, d)


# speedup vs baseline: 4.0961x; 2.4109x over previous
import functools

import jax
import jax.numpy as jnp
from jax import lax
from jax.experimental import pallas as pl
from jax.experimental.pallas import tpu as pltpu
from jax.experimental.pallas import tpu_sc as plsc

_LANES = 16
_CHUNK_ROWS = 8
_N_IN = 3
_N_OUT = 2


@functools.lru_cache(maxsize=None)
def _make_sc_add(batch: int, seq_len: int, d: int, nc: int, ns: int):
    nw = nc * ns
    assert seq_len % nw == 0
    s_per_w = seq_len // nw
    chunk = min(_CHUNK_ROWS, s_per_w)
    assert s_per_w % chunk == 0
    cpb = s_per_w // chunk
    n_chunks = batch * cpb
    assert d % _LANES == 0
    vec_per_row = d // _LANES
    n_vec = chunk * vec_per_row
    n_in = min(_N_IN, n_chunks)
    n_out = min(_N_OUT, n_chunks)

    mesh = plsc.VectorSubcoreMesh(
        core_axis_name="c", subcore_axis_name="s",
        num_cores=nc, num_subcores=ns)

    @functools.partial(
        pl.kernel,
        out_type=jax.ShapeDtypeStruct((batch, seq_len, d), jnp.float32),
        mesh=mesh,
        compiler_params=pltpu.CompilerParams(use_tc_tiling_on_sc=True),
        scratch_types=(
            [pltpu.VMEM((s_per_w, d), jnp.float32)]
            + [pltpu.VMEM((chunk, d), jnp.float32) for _ in range(n_in)]
            + [pltpu.VMEM((chunk, d), jnp.float32) for _ in range(n_out)]
            + [pltpu.SemaphoreType.DMA for _ in range(n_in + n_out)]
        ),
    )
    def sc_add(x_hbm, pe_hbm, out_hbm, *scratch):
        pv = scratch[0]
        xv = scratch[1:1 + n_in]
        ov = scratch[1 + n_in:1 + n_in + n_out]
        in_sem = scratch[1 + n_in + n_out:1 + 2 * n_in + n_out]
        out_sem = scratch[1 + 2 * n_in + n_out:]

        wid = lax.axis_index("s") * nc + lax.axis_index("c")
        seq_base = wid * s_per_w

        pltpu.sync_copy(pe_hbm.at[pl.ds(seq_base, s_per_w), :], pv)

        def x_slice(t):
            b, c = divmod(t, cpb)
            return (b, pl.ds(seq_base + c * chunk, chunk))

        in_d = [None] * n_chunks
        out_d = [None] * n_chunks
        for t in range(n_in):
            b, rows = x_slice(t)
            in_d[t] = pltpu.async_copy(
                x_hbm.at[b, rows, :], xv[t % n_in], in_sem[t % n_in])

        for t in range(n_chunks):
            ib = t % n_in
            ob = t % n_out
            in_d[t].wait()
            if t - n_out >= 0:
                out_d[t - n_out].wait()

            row_base = (t % cpb) * chunk

            @plsc.parallel_loop(0, n_vec, unroll=8)
            def _add(i, _ib=ib, _ob=ob, _rb=row_base):
                r = i // vec_per_row
                o = (i % vec_per_row) * _LANES
                ov[_ob][r, pl.ds(o, _LANES)] = (
                    xv[_ib][r, pl.ds(o, _LANES)] + pv[_rb + r, pl.ds(o, _LANES)])

            b, rows = x_slice(t)
            out_d[t] = pltpu.async_copy(
                ov[ob], out_hbm.at[b, rows, :], out_sem[ob])
            if t + n_in < n_chunks:
                b2, rows2 = x_slice(t + n_in)
                in_d[t + n_in] = pltpu.async_copy(
                    x_hbm.at[b2, rows2, :], xv[ib], in_sem[ib])

        for t in range(max(0, n_chunks - n_out), n_chunks):
            out_d[t].wait()

    return sc_add


def kernel(x, pos_embedding):
    b, s, d = x.shape
    info = plsc.get_sparse_core_info()
    fn = _make_sc_add(b, s, d, info.num_cores, info.num_subcores)
    return fn(x, pos_embedding)


# speedup vs baseline: 4.3815x; 1.0697x over previous
import functools

import jax
import jax.numpy as jnp
from jax import lax
from jax.experimental import pallas as pl
from jax.experimental.pallas import tpu as pltpu
from jax.experimental.pallas import tpu_sc as plsc

_LANES = 16
_CHUNK_ROWS = 16
_N_IN = 3
_N_OUT = 2
_N_PE = 2


@functools.lru_cache(maxsize=None)
def _make_sc_add(batch: int, seq_len: int, d: int, nc: int, ns: int):
    nw = nc * ns
    assert seq_len % nw == 0
    s_per_w = seq_len // nw
    chunk = min(_CHUNK_ROWS, s_per_w)
    assert s_per_w % chunk == 0
    cpb = s_per_w // chunk
    n_chunks = batch * cpb
    assert d % _LANES == 0
    vec_per_row = d // _LANES
    n_vec = chunk * vec_per_row
    n_in = min(_N_IN, n_chunks)
    n_out = min(_N_OUT, n_chunks)
    n_pe = min(_N_PE, cpb)

    mesh = plsc.VectorSubcoreMesh(
        core_axis_name="c", subcore_axis_name="s",
        num_cores=nc, num_subcores=ns)

    @functools.partial(
        pl.kernel,
        out_type=jax.ShapeDtypeStruct((batch, seq_len, d), jnp.float32),
        mesh=mesh,
        compiler_params=pltpu.CompilerParams(use_tc_tiling_on_sc=True),
        scratch_types=(
            [pltpu.VMEM((chunk, d), jnp.float32) for _ in range(n_pe)]
            + [pltpu.VMEM((chunk, d), jnp.float32) for _ in range(n_in)]
            + [pltpu.VMEM((chunk, d), jnp.float32) for _ in range(n_out)]
            + [pltpu.SemaphoreType.DMA for _ in range(n_pe + n_in + n_out)]
        ),
    )
    def sc_add(x_hbm, pe_hbm, out_hbm, *scratch):
        pv = scratch[:n_pe]
        xv = scratch[n_pe:n_pe + n_in]
        ov = scratch[n_pe + n_in:n_pe + n_in + n_out]
        sems = scratch[n_pe + n_in + n_out:]
        pe_sem = sems[:n_pe]
        in_sem = sems[n_pe:n_pe + n_in]
        out_sem = sems[n_pe + n_in:]

        wid = lax.axis_index("s") * nc + lax.axis_index("c")
        seq_base = wid * s_per_w

        def rows_of(t):
            c = t // batch
            return pl.ds(seq_base + c * chunk, chunk)

        def x_slice(t):
            return (t % batch, rows_of(t))

        in_d = [None] * n_chunks
        out_d = [None] * n_chunks
        pe_d = [None] * cpb

        for c in range(n_pe):
            pe_d[c] = pltpu.async_copy(
                pe_hbm.at[pl.ds(seq_base + c * chunk, chunk), :],
                pv[c % n_pe], pe_sem[c % n_pe])
        for t in range(n_in):
            b, rows = x_slice(t)
            in_d[t] = pltpu.async_copy(
                x_hbm.at[b, rows, :], xv[t % n_in], in_sem[t % n_in])

        for t in range(n_chunks):
            ib = t % n_in
            ob = t % n_out
            c = t // batch
            pb = c % n_pe
            if t % batch == 0:
                pe_d[c].wait()
            in_d[t].wait()
            if t - n_out >= 0:
                out_d[t - n_out].wait()

            @plsc.parallel_loop(0, n_vec, unroll=8)
            def _add(i, _ib=ib, _ob=ob, _pb=pb):
                r = i // vec_per_row
                o = (i % vec_per_row) * _LANES
                ov[_ob][r, pl.ds(o, _LANES)] = (
                    xv[_ib][r, pl.ds(o, _LANES)] + pv[_pb][r, pl.ds(o, _LANES)])

            b, rows = x_slice(t)
            out_d[t] = pltpu.async_copy(
                ov[ob], out_hbm.at[b, rows, :], out_sem[ob])
            if t + n_in < n_chunks:
                b2, rows2 = x_slice(t + n_in)
                in_d[t + n_in] = pltpu.async_copy(
                    x_hbm.at[b2, rows2, :], xv[ib], in_sem[ib])
            if t % batch == batch - 1:
                cn = c + n_pe
                if cn < cpb:
                    pe_d[cn] = pltpu.async_copy(
                        pe_hbm.at[pl.ds(seq_base + cn * chunk, chunk), :],
                        pv[cn % n_pe], pe_sem[cn % n_pe])

        for t in range(max(0, n_chunks - n_out), n_chunks):
            out_d[t].wait()

    return sc_add


def kernel(x, pos_embedding):
    b, s, d = x.shape
    info = plsc.get_sparse_core_info()
    fn = _make_sc_add(b, s, d, info.num_cores, info.num_subcores)
    return fn(x, pos_embedding)
